# baseline (speedup 1.0000x reference)
import jax
import jax.numpy as jnp
from jax.experimental import pallas as pl
from jax.experimental.pallas import tpu as pltpu

_BLOCK_ROWS = 2000


def _mlp_body(x_ref, p_ref, o_ref):
    w1 = p_ref[0:128, :]
    w2 = p_ref[128:256, :]
    b1 = p_ref[256:257, :]
    b2 = p_ref[264:265, :]
    h = jnp.dot(x_ref[:].astype(jnp.bfloat16), w1.astype(jnp.bfloat16),
                preferred_element_type=jnp.float32)
    h = h + b1
    h = jnp.where(h > 0, h, jnp.exp(h) - 1.0)
    o = jnp.dot(h.astype(jnp.bfloat16), w2.astype(jnp.bfloat16),
                preferred_element_type=jnp.float32)
    o_ref[:] = o + b2


def kernel(x, edge_index, W1, b1, W2, b2):
    del edge_index
    N, D = x.shape
    pad = jnp.zeros((7, D), dtype=x.dtype)
    packed = jnp.concatenate(
        [W1, W2, b1.reshape(1, D), pad, b2.reshape(1, D), pad], axis=0)
    return pl.pallas_call(
        _mlp_body,
        grid=(N // _BLOCK_ROWS,),
        in_specs=[
            pl.BlockSpec((_BLOCK_ROWS, D), lambda i: (i, 0)),
            pl.BlockSpec((272, D), lambda i: (0, 0)),
        ],
        out_specs=pl.BlockSpec((_BLOCK_ROWS, D), lambda i: (i, 0)),
        out_shape=jax.ShapeDtypeStruct((N, D), x.dtype),
        compiler_params=pltpu.CompilerParams(
            dimension_semantics=("arbitrary",)),
    )(x, packed)


# speedup vs baseline: 2.0878x; 2.0878x over previous
import jax
import jax.numpy as jnp
from jax.experimental import pallas as pl
from jax.experimental.pallas import tpu as pltpu

_BLOCK_ROWS = 4000


def _mlp_body(x_ref, w1_ref, b1_ref, w2_ref, b2_ref, o_ref):
    h = jnp.dot(x_ref[:], w1_ref[:],
                preferred_element_type=jnp.float32)
    h = h + b1_ref[:]
    h = jnp.where(h > 0, h, jnp.exp(h) - 1.0)
    o = jnp.dot(h, w2_ref[:],
                preferred_element_type=jnp.float32)
    o_ref[:] = o + b2_ref[:]


def kernel(x, edge_index, W1, b1, W2, b2):
    del edge_index
    N, D = x.shape
    b1_2d = b1.reshape(1, D)
    b2_2d = b2.reshape(1, D)
    grid = (pl.cdiv(N, _BLOCK_ROWS),)
    return pl.pallas_call(
        _mlp_body,
        grid=grid,
        in_specs=[
            pl.BlockSpec((_BLOCK_ROWS, D), lambda i: (i, 0)),
            pl.BlockSpec((D, D), lambda i: (0, 0)),
            pl.BlockSpec((1, D), lambda i: (0, 0)),
            pl.BlockSpec((D, D), lambda i: (0, 0)),
            pl.BlockSpec((1, D), lambda i: (0, 0)),
        ],
        out_specs=pl.BlockSpec((_BLOCK_ROWS, D), lambda i: (i, 0)),
        out_shape=jax.ShapeDtypeStruct((N, D), x.dtype),
        compiler_params=pltpu.CompilerParams(
            dimension_semantics=("arbitrary",)),
    )(x, W1, b1_2d, W2, b2_2d)
